# baseline (speedup 1.0000x reference)
import functools
import math

import jax
import jax.numpy as jnp
import numpy as np
from jax import lax
from jax.experimental import pallas as pl
from jax.experimental.pallas import tpu as pltpu
from jax.experimental.pallas import tpu_sc as plsc

VOCAB_SIZE = 32000
MODEL_DIM = 512
MAX_SEQ_LENGTH = 2048
SCALE = math.sqrt(MODEL_DIM)

NUM_CORES = 2
NUM_SUBCORES = 16
NUM_WORKERS = NUM_CORES * NUM_SUBCORES

BATCH = 4
SEQ = 2048
TOTAL_ROWS = BATCH * SEQ
PIECES = (4096, 2048, 2048)

ROW_BLOCK = 2048


def _pos_encoding_np(max_seq_length, model_dim):
    position = np.arange(max_seq_length)[:, None].astype(np.float32)
    div_term = np.exp(
        np.arange(0, model_dim, 2).astype(np.float32)
        * (-math.log(10000.0) / model_dim)
    )
    pe = np.zeros((max_seq_length, model_dim), dtype=np.float32)
    pe[:, 0::2] = np.sin(position * div_term)
    pe[:, 1::2] = np.cos(position * div_term)
    return pe


_PE = _pos_encoding_np(MAX_SEQ_LENGTH, MODEL_DIM)


def _sc_gather_body(half_base, rows_per_worker, chunk_rows,
                    idx_hbm, table_hbm, out_hbm, idx_v, rows0, rows1,
                    sem_g0, sem_g1, sem_s0, sem_s1):
    num_chunks = rows_per_worker // chunk_rows
    wid = lax.axis_index("s") * NUM_CORES + lax.axis_index("c")
    base = wid * rows_per_worker

    rows = [rows0, rows1]
    sem_g = [sem_g0, sem_g1]
    sem_s = [sem_s0, sem_s1]

    pltpu.sync_copy(idx_hbm.at[pl.ds(half_base + base, rows_per_worker)], idx_v)

    def fire_gather(c):
        b = c % 2
        return pltpu.async_copy(
            table_hbm.at[idx_v.at[pl.ds(c * chunk_rows, chunk_rows)]],
            rows[b], sem_g[b])

    pending = {0: fire_gather(0)}
    stores = {}

    for c in range(num_chunks):
        b = c % 2
        if c - 1 in stores:
            stores.pop(c - 1).wait()
        if c + 1 < num_chunks:
            pending[c + 1] = fire_gather(c + 1)
        pending.pop(c).wait()
        stores[c] = pltpu.async_copy(
            rows[b], out_hbm.at[pl.ds(base + c * chunk_rows, chunk_rows)],
            sem_s[b])

    for c in sorted(stores):
        stores[c].wait()


def _fma_kernel(g_ref, pe_ref, o_ref):
    o_ref[...] = g_ref[...] * SCALE + pe_ref[...]


def _fma_rest_kernel(g_ref, pe_ref, _, o_ref):
    o_ref[...] = g_ref[...] * SCALE + pe_ref[...]


def _sc_gather(x_flat, table, half_base, num_rows):
    rows_per_worker = num_rows // NUM_WORKERS
    chunk_rows = rows_per_worker // 2
    mesh = plsc.VectorSubcoreMesh(
        core_axis_name="c", subcore_axis_name="s",
        num_cores=NUM_CORES, num_subcores=NUM_SUBCORES)
    return pl.kernel(
        functools.partial(_sc_gather_body, half_base, rows_per_worker,
                          chunk_rows),
        out_type=jax.ShapeDtypeStruct((num_rows, MODEL_DIM), jnp.float32),
        mesh=mesh,
        scratch_types=[
            pltpu.VMEM((rows_per_worker,), jnp.int32),
            pltpu.VMEM((chunk_rows, MODEL_DIM), jnp.float32),
            pltpu.VMEM((chunk_rows, MODEL_DIM), jnp.float32),
            pltpu.SemaphoreType.DMA,
            pltpu.SemaphoreType.DMA,
            pltpu.SemaphoreType.DMA,
            pltpu.SemaphoreType.DMA,
        ],
    )(x_flat, table)


@jax.jit
def _embed(x, table):
    x_flat = x.reshape(TOTAL_ROWS).astype(jnp.int32)
    pe = jnp.asarray(_PE)

    gathered = [
        _sc_gather(x_flat, table, base, size)
        for base, size in zip(_piece_bases(), PIECES)
    ]

    partial = None
    for (base, size), g in zip(zip(_piece_bases(), PIECES), gathered):
        n0 = base // ROW_BLOCK
        if partial is None:
            partial = pl.pallas_call(
                _fma_kernel,
                out_shape=jax.ShapeDtypeStruct(
                    (TOTAL_ROWS, MODEL_DIM), jnp.float32),
                grid=(size // ROW_BLOCK,),
                in_specs=[
                    pl.BlockSpec((ROW_BLOCK, MODEL_DIM), lambda i: (i, 0)),
                    pl.BlockSpec((ROW_BLOCK, MODEL_DIM), lambda i: (0, 0)),
                ],
                out_specs=pl.BlockSpec(
                    (ROW_BLOCK, MODEL_DIM),
                    functools.partial(lambda n, i: (i + n, 0), n0)),
                compiler_params=pltpu.CompilerParams(
                    dimension_semantics=("arbitrary",),
                ),
            )(g, pe)
        else:
            partial = pl.pallas_call(
                _fma_rest_kernel,
                out_shape=jax.ShapeDtypeStruct(
                    (TOTAL_ROWS, MODEL_DIM), jnp.float32),
                grid=(size // ROW_BLOCK,),
                in_specs=[
                    pl.BlockSpec((ROW_BLOCK, MODEL_DIM), lambda i: (i, 0)),
                    pl.BlockSpec((ROW_BLOCK, MODEL_DIM), lambda i: (0, 0)),
                    pl.BlockSpec(memory_space=pl.ANY),
                ],
                out_specs=pl.BlockSpec(
                    (ROW_BLOCK, MODEL_DIM),
                    functools.partial(lambda n, i: (i + n, 0), n0)),
                input_output_aliases={2: 0},
                compiler_params=pltpu.CompilerParams(
                    dimension_semantics=("arbitrary",),
                ),
            )(g, pe, partial)
    return partial.reshape(BATCH, SEQ, MODEL_DIM)


def _piece_bases():
    bases = []
    acc = 0
    for size in PIECES:
        bases.append(acc)
        acc += size
    return tuple(bases)


def kernel(x, table):
    return _embed(x, table)


# speedup vs baseline: 1.1185x; 1.1185x over previous
import functools
import math

import jax
import jax.numpy as jnp
import numpy as np
from jax import lax
from jax.experimental import pallas as pl
from jax.experimental.pallas import tpu as pltpu
from jax.experimental.pallas import tpu_sc as plsc

VOCAB_SIZE = 32000
MODEL_DIM = 512
MAX_SEQ_LENGTH = 2048
SCALE = math.sqrt(MODEL_DIM)

NUM_CORES = 2
NUM_SUBCORES = 16
NUM_WORKERS = NUM_CORES * NUM_SUBCORES

BATCH = 4
SEQ = 2048
TOTAL_ROWS = BATCH * SEQ
HALF_ROWS = TOTAL_ROWS // 2
ROWS_PER_WORKER = HALF_ROWS // NUM_WORKERS
CHUNK_ROWS = 64
NUM_CHUNKS = ROWS_PER_WORKER // CHUNK_ROWS

ROW_BLOCK = 2048


def _pos_encoding_np(max_seq_length, model_dim):
    position = np.arange(max_seq_length)[:, None].astype(np.float32)
    div_term = np.exp(
        np.arange(0, model_dim, 2).astype(np.float32)
        * (-math.log(10000.0) / model_dim)
    )
    pe = np.zeros((max_seq_length, model_dim), dtype=np.float32)
    pe[:, 0::2] = np.sin(position * div_term)
    pe[:, 1::2] = np.cos(position * div_term)
    return pe


_PE = _pos_encoding_np(MAX_SEQ_LENGTH, MODEL_DIM)


def _sc_gather_body(half_base, idx_hbm, table_hbm, out_hbm, idx_v, rows0, rows1,
                    sem_g0, sem_g1, sem_s0, sem_s1):
    wid = lax.axis_index("s") * NUM_CORES + lax.axis_index("c")
    base = wid * ROWS_PER_WORKER
    flat = half_base + base
    b = flat // SEQ
    col = flat % SEQ

    rows = [rows0, rows1]
    sem_g = [sem_g0, sem_g1]
    sem_s = [sem_s0, sem_s1]

    pltpu.sync_copy(idx_hbm.at[b, pl.ds(col, ROWS_PER_WORKER)], idx_v)

    def fire_gather(c):
        buf = c % 2
        return pltpu.async_copy(
            table_hbm.at[idx_v.at[pl.ds(c * CHUNK_ROWS, CHUNK_ROWS)]],
            rows[buf], sem_g[buf])

    pending = {0: fire_gather(0)}
    stores = {}

    for c in range(NUM_CHUNKS):
        buf = c % 2
        if c - 1 in stores:
            stores.pop(c - 1).wait()
        if c + 1 < NUM_CHUNKS:
            pending[c + 1] = fire_gather(c + 1)
        pending.pop(c).wait()
        stores[c] = pltpu.async_copy(
            rows[buf], out_hbm.at[pl.ds(base + c * CHUNK_ROWS, CHUNK_ROWS)],
            sem_s[buf])

    for c in sorted(stores):
        stores[c].wait()


def _fma_kernel(g_ref, pe_ref, o_ref):
    o_ref[...] = g_ref[...] * SCALE + pe_ref[...]


def _fma_rest_kernel(g_ref, pe_ref, _, o_ref):
    o_ref[...] = g_ref[...] * SCALE + pe_ref[...]


def _sc_gather(x, table, half_base):
    mesh = plsc.VectorSubcoreMesh(
        core_axis_name="c", subcore_axis_name="s",
        num_cores=NUM_CORES, num_subcores=NUM_SUBCORES)
    return pl.kernel(
        functools.partial(_sc_gather_body, half_base),
        out_type=jax.ShapeDtypeStruct((HALF_ROWS, MODEL_DIM), jnp.float32),
        mesh=mesh,
        scratch_types=[
            pltpu.VMEM((ROWS_PER_WORKER,), jnp.int32),
            pltpu.VMEM((CHUNK_ROWS, MODEL_DIM), jnp.float32),
            pltpu.VMEM((CHUNK_ROWS, MODEL_DIM), jnp.float32),
            pltpu.SemaphoreType.DMA,
            pltpu.SemaphoreType.DMA,
            pltpu.SemaphoreType.DMA,
            pltpu.SemaphoreType.DMA,
        ],
    )(x, table)


@jax.jit
def _embed(x, table):
    xi = x.astype(jnp.int32)
    pe = jnp.asarray(_PE)

    g0 = _sc_gather(xi, table, 0)
    g1 = _sc_gather(xi, table, HALF_ROWS)

    partial = pl.pallas_call(
        _fma_kernel,
        out_shape=jax.ShapeDtypeStruct((TOTAL_ROWS, MODEL_DIM), jnp.float32),
        grid=(HALF_ROWS // ROW_BLOCK,),
        in_specs=[
            pl.BlockSpec((ROW_BLOCK, MODEL_DIM), lambda i: (i, 0)),
            pl.BlockSpec((ROW_BLOCK, MODEL_DIM), lambda i: (0, 0)),
        ],
        out_specs=pl.BlockSpec((ROW_BLOCK, MODEL_DIM), lambda i: (i, 0)),
        compiler_params=pltpu.CompilerParams(
            dimension_semantics=("arbitrary",),
        ),
    )(g0, pe)

    n0 = HALF_ROWS // ROW_BLOCK
    out = pl.pallas_call(
        _fma_rest_kernel,
        out_shape=jax.ShapeDtypeStruct((TOTAL_ROWS, MODEL_DIM), jnp.float32),
        grid=(HALF_ROWS // ROW_BLOCK,),
        in_specs=[
            pl.BlockSpec((ROW_BLOCK, MODEL_DIM), lambda i: (i, 0)),
            pl.BlockSpec((ROW_BLOCK, MODEL_DIM), lambda i: (0, 0)),
            pl.BlockSpec(memory_space=pl.ANY),
        ],
        out_specs=pl.BlockSpec(
            (ROW_BLOCK, MODEL_DIM), lambda i: (i + n0, 0)),
        input_output_aliases={2: 0},
        compiler_params=pltpu.CompilerParams(
            dimension_semantics=("arbitrary",),
        ),
    )(g1, pe, partial)
    return out.reshape(BATCH, SEQ, MODEL_DIM)


def kernel(x, table):
    return _embed(x, table)


# speedup vs baseline: 1.1356x; 1.0153x over previous
import functools
import math

import jax
import jax.numpy as jnp
import numpy as np
from jax import lax
from jax.experimental import pallas as pl
from jax.experimental.pallas import tpu as pltpu
from jax.experimental.pallas import tpu_sc as plsc

VOCAB_SIZE = 32000
MODEL_DIM = 512
MAX_SEQ_LENGTH = 2048
SCALE = math.sqrt(MODEL_DIM)

NUM_CORES = 2
NUM_SUBCORES = 16
NUM_WORKERS = NUM_CORES * NUM_SUBCORES

BATCH = 4
SEQ = 2048
TOTAL_ROWS = BATCH * SEQ
ROWS_PER_WORKER = TOTAL_ROWS // NUM_WORKERS
CHUNK_ROWS = 64
NUM_CHUNKS = ROWS_PER_WORKER // CHUNK_ROWS

ROW_BLOCK = 2048


def _pos_encoding_np(max_seq_length, model_dim):
    position = np.arange(max_seq_length)[:, None].astype(np.float32)
    div_term = np.exp(
        np.arange(0, model_dim, 2).astype(np.float32)
        * (-math.log(10000.0) / model_dim)
    )
    pe = np.zeros((max_seq_length, model_dim), dtype=np.float32)
    pe[:, 0::2] = np.sin(position * div_term)
    pe[:, 1::2] = np.cos(position * div_term)
    return pe


_PE = _pos_encoding_np(MAX_SEQ_LENGTH, MODEL_DIM)


def _sc_gather_body(idx_hbm, table_hbm, out_hbm, idx_v, rows0, rows1,
                    sem_g0, sem_g1, sem_s0, sem_s1):
    wid = lax.axis_index("s") * NUM_CORES + lax.axis_index("c")
    base = wid * ROWS_PER_WORKER
    b = base // SEQ
    col = base % SEQ

    rows = [rows0, rows1]
    sem_g = [sem_g0, sem_g1]
    sem_s = [sem_s0, sem_s1]

    pltpu.sync_copy(idx_hbm.at[b, pl.ds(col, ROWS_PER_WORKER)], idx_v)

    def fire_gather(c):
        buf = c % 2
        return pltpu.async_copy(
            table_hbm.at[idx_v.at[pl.ds(c * CHUNK_ROWS, CHUNK_ROWS)]],
            rows[buf], sem_g[buf])

    pending = {0: fire_gather(0)}
    stores = {}

    for c in range(NUM_CHUNKS):
        buf = c % 2
        if c - 1 in stores:
            stores.pop(c - 1).wait()
        if c + 1 < NUM_CHUNKS:
            pending[c + 1] = fire_gather(c + 1)
        pending.pop(c).wait()
        stores[c] = pltpu.async_copy(
            rows[buf], out_hbm.at[pl.ds(base + c * CHUNK_ROWS, CHUNK_ROWS)],
            sem_s[buf])

    for c in sorted(stores):
        stores[c].wait()


def _fma_kernel(g_ref, pe_ref, o_ref):
    o_ref[...] = g_ref[...] * SCALE + pe_ref[...]


def _fma_rest_kernel(g_ref, pe_ref, _, o_ref):
    o_ref[...] = g_ref[...] * SCALE + pe_ref[...]


def _sc_gather(x, table):
    mesh = plsc.VectorSubcoreMesh(
        core_axis_name="c", subcore_axis_name="s",
        num_cores=NUM_CORES, num_subcores=NUM_SUBCORES)
    return pl.kernel(
        _sc_gather_body,
        out_type=jax.ShapeDtypeStruct((TOTAL_ROWS, MODEL_DIM), jnp.float32),
        mesh=mesh,
        scratch_types=[
            pltpu.VMEM((ROWS_PER_WORKER,), jnp.int32),
            pltpu.VMEM((CHUNK_ROWS, MODEL_DIM), jnp.float32),
            pltpu.VMEM((CHUNK_ROWS, MODEL_DIM), jnp.float32),
            pltpu.SemaphoreType.DMA,
            pltpu.SemaphoreType.DMA,
            pltpu.SemaphoreType.DMA,
            pltpu.SemaphoreType.DMA,
        ],
    )(x, table)


@jax.jit
def _embed(x, table):
    xi = x.astype(jnp.int32)
    pe = jnp.asarray(_PE)

    g = _sc_gather(xi, table)

    out = pl.pallas_call(
        _fma_kernel,
        out_shape=jax.ShapeDtypeStruct((TOTAL_ROWS, MODEL_DIM), jnp.float32),
        grid=(TOTAL_ROWS // ROW_BLOCK,),
        in_specs=[
            pl.BlockSpec((ROW_BLOCK, MODEL_DIM), lambda i: (i, 0)),
            pl.BlockSpec((ROW_BLOCK, MODEL_DIM), lambda i: (0, 0)),
        ],
        out_specs=pl.BlockSpec((ROW_BLOCK, MODEL_DIM), lambda i: (i, 0)),
        compiler_params=pltpu.CompilerParams(
            dimension_semantics=("arbitrary",),
        ),
    )(g, pe)
    return out.reshape(BATCH, SEQ, MODEL_DIM)


def kernel(x, table):
    return _embed(x, table)
